# baseline (speedup 1.0000x reference)
import jax
import jax.numpy as jnp
from jax.experimental import pallas as pl
from jax.experimental.pallas import tpu as pltpu

N_TOKENS = 16384
IN_DIM = 1024
HIDDEN = 256
NUM_EXPERTS = 16

BM = 2048
KSPLIT = 512


def _dot(a, b):
    return jax.lax.dot_general(
        a, b, (((1,), (0,)), ((), ())), preferred_element_type=jnp.float32
    )


def _router_body(xa_ref, xb_ref, w1_ref, b1_ref, w2_ref, b2_ref,
                 probs_ref, eid_ref):
    w1 = w1_ref[...].astype(jnp.bfloat16)
    ha = _dot(xa_ref[...].astype(jnp.bfloat16), w1[:KSPLIT])
    hb = _dot(xb_ref[...].astype(jnp.bfloat16), w1[KSPLIT:])
    h = jnp.maximum(ha + hb + b1_ref[...], 0.0)
    logits = _dot(h.astype(jnp.bfloat16), w2_ref[...].astype(jnp.bfloat16))
    logits = logits + b2_ref[...]
    m = jnp.max(logits, axis=-1, keepdims=True)
    e = jnp.exp(logits - m)
    probs_ref[...] = e / jnp.sum(e, axis=-1, keepdims=True)
    eid_ref[...] = jnp.argmax(logits, axis=-1).astype(jnp.int32)


def kernel(patch_feat, W1, b1, W2, b2):
    b1_2d = b1.reshape(1, HIDDEN)
    b2_2d = b2.reshape(1, NUM_EXPERTS)
    grid = (N_TOKENS // BM,)
    probs, eid = pl.pallas_call(
        _router_body,
        grid=grid,
        in_specs=[
            pl.BlockSpec((BM, KSPLIT), lambda i: (i, 0)),
            pl.BlockSpec((BM, KSPLIT), lambda i: (i, 1)),
            pl.BlockSpec((IN_DIM, HIDDEN), lambda i: (0, 0)),
            pl.BlockSpec((1, HIDDEN), lambda i: (0, 0)),
            pl.BlockSpec((HIDDEN, NUM_EXPERTS), lambda i: (0, 0)),
            pl.BlockSpec((1, NUM_EXPERTS), lambda i: (0, 0)),
        ],
        out_specs=[
            pl.BlockSpec((BM, NUM_EXPERTS), lambda i: (i, 0)),
            pl.BlockSpec((BM,), lambda i: (i,)),
        ],
        out_shape=[
            jax.ShapeDtypeStruct((N_TOKENS, NUM_EXPERTS), jnp.float32),
            jax.ShapeDtypeStruct((N_TOKENS,), jnp.int32),
        ],
        compiler_params=pltpu.CompilerParams(
            dimension_semantics=("parallel",),
        ),
    )(patch_feat, patch_feat, W1, b1_2d, W2, b2_2d)
    return probs, eid


# speedup vs baseline: 1.3271x; 1.3271x over previous
import jax
import jax.numpy as jnp
from jax.experimental import pallas as pl
from jax.experimental.pallas import tpu as pltpu

N_TOKENS = 16384
IN_DIM = 1024
HIDDEN = 256
NUM_EXPERTS = 16

BM = 2048
KSPLIT = 512


def _dot(a, b):
    return jax.lax.dot_general(
        a, b, (((1,), (0,)), ((), ())), preferred_element_type=jnp.float32
    )


def _router_body(xa_ref, xb_ref, w1a_ref, w1b_ref, b1_ref, w2_ref, b2_ref,
                 probs_ref, eid_ref):
    ha = _dot(xa_ref[...].astype(jnp.bfloat16), w1a_ref[...])
    hb = _dot(xb_ref[...].astype(jnp.bfloat16), w1b_ref[...])
    h = jnp.maximum(ha + hb + b1_ref[...], 0.0)
    logits = _dot(h.astype(jnp.bfloat16), w2_ref[...])
    logits = logits + b2_ref[...]
    m = jnp.max(logits, axis=-1, keepdims=True)
    e = jnp.exp(logits - m)
    probs_ref[...] = e / jnp.sum(e, axis=-1, keepdims=True)

    mask = (logits == m).astype(jnp.bfloat16)
    lt = (
        jax.lax.broadcasted_iota(jnp.int32, (NUM_EXPERTS, NUM_EXPERTS), 0)
        <= jax.lax.broadcasted_iota(jnp.int32, (NUM_EXPERTS, NUM_EXPERTS), 1)
    ).astype(jnp.bfloat16)
    cnt = _dot(mask, lt)
    first = jnp.where(cnt == 1.0, mask.astype(jnp.float32), 0.0).astype(jnp.bfloat16)
    first_t = jax.lax.transpose(first, (1, 0))
    iota_row = jax.lax.broadcasted_iota(
        jnp.int32, (1, NUM_EXPERTS), 1
    ).astype(jnp.bfloat16)
    eid_lane = _dot(iota_row, first_t)
    eid_ref[...] = eid_lane.astype(jnp.int32).reshape(BM // 256, 256)


def kernel(patch_feat, W1, b1, W2, b2):
    w1a = W1[:KSPLIT].astype(jnp.bfloat16)
    w1b = W1[KSPLIT:].astype(jnp.bfloat16)
    w2 = W2.astype(jnp.bfloat16)
    b1_2d = b1.reshape(1, HIDDEN)
    b2_2d = b2.reshape(1, NUM_EXPERTS)
    grid = (N_TOKENS // BM,)
    probs, eid = pl.pallas_call(
        _router_body,
        grid=grid,
        in_specs=[
            pl.BlockSpec((BM, KSPLIT), lambda i: (i, 0)),
            pl.BlockSpec((BM, KSPLIT), lambda i: (i, 1)),
            pl.BlockSpec((KSPLIT, HIDDEN), lambda i: (0, 0)),
            pl.BlockSpec((KSPLIT, HIDDEN), lambda i: (0, 0)),
            pl.BlockSpec((1, HIDDEN), lambda i: (0, 0)),
            pl.BlockSpec((HIDDEN, NUM_EXPERTS), lambda i: (0, 0)),
            pl.BlockSpec((1, NUM_EXPERTS), lambda i: (0, 0)),
        ],
        out_specs=[
            pl.BlockSpec((BM, NUM_EXPERTS), lambda i: (i, 0)),
            pl.BlockSpec((BM // 256, 256), lambda i: (i, 0)),
        ],
        out_shape=[
            jax.ShapeDtypeStruct((N_TOKENS, NUM_EXPERTS), jnp.float32),
            jax.ShapeDtypeStruct((N_TOKENS // 256, 256), jnp.int32),
        ],
        compiler_params=pltpu.CompilerParams(
            dimension_semantics=("parallel",),
        ),
    )(patch_feat, patch_feat, w1a, w1b, b1_2d, w2, b2_2d)
    return probs, eid.reshape(N_TOKENS)


# speedup vs baseline: 1.4040x; 1.0580x over previous
import jax
import jax.numpy as jnp
from jax.experimental import pallas as pl
from jax.experimental.pallas import tpu as pltpu

N_TOKENS = 16384
IN_DIM = 1024
HIDDEN = 256
NUM_EXPERTS = 16

BM = 2048


def _dot(a, b):
    return jax.lax.dot_general(
        a, b, (((1,), (0,)), ((), ())), preferred_element_type=jnp.float32
    )


def _router_body(x_ref, w1_ref, b1_ref, w2_ref, b2_ref, probs_ref, eid_ref):
    w1 = w1_ref[...].astype(jnp.bfloat16)
    b1 = b1_ref[...].reshape(1, HIDDEN)
    w2 = w2_ref[...].astype(jnp.bfloat16)
    b2 = b2_ref[...].reshape(1, NUM_EXPERTS)
    h = _dot(x_ref[...].astype(jnp.bfloat16), w1)
    h = jnp.maximum(h + b1, 0.0)
    logits = _dot(h.astype(jnp.bfloat16), w2)
    logits = logits + b2
    m = jnp.max(logits, axis=-1, keepdims=True)
    e = jnp.exp(logits - m)
    probs_ref[...] = e / jnp.sum(e, axis=-1, keepdims=True)

    mask = (logits == m).astype(jnp.bfloat16)
    lt = (
        jax.lax.broadcasted_iota(jnp.int32, (NUM_EXPERTS, NUM_EXPERTS), 0)
        <= jax.lax.broadcasted_iota(jnp.int32, (NUM_EXPERTS, NUM_EXPERTS), 1)
    ).astype(jnp.bfloat16)
    cnt = _dot(mask, lt)
    first = jnp.where(cnt == 1.0, mask.astype(jnp.float32), 0.0).astype(jnp.bfloat16)
    first_t = jax.lax.transpose(first, (1, 0))
    iota_row = jax.lax.broadcasted_iota(
        jnp.int32, (1, NUM_EXPERTS), 1
    ).astype(jnp.bfloat16)
    eid_lane = _dot(iota_row, first_t)
    eid_ref[...] = eid_lane.astype(jnp.int32).reshape(BM // 256, 256)


def kernel(patch_feat, W1, b1, W2, b2):
    grid = (N_TOKENS // BM,)
    probs, eid = pl.pallas_call(
        _router_body,
        grid=grid,
        in_specs=[
            pl.BlockSpec((BM, IN_DIM), lambda i: (i, 0)),
            pl.BlockSpec((IN_DIM, HIDDEN), lambda i: (0, 0)),
            pl.BlockSpec((HIDDEN,), lambda i: (0,)),
            pl.BlockSpec((HIDDEN, NUM_EXPERTS), lambda i: (0, 0)),
            pl.BlockSpec((NUM_EXPERTS,), lambda i: (0,)),
        ],
        out_specs=[
            pl.BlockSpec((BM, NUM_EXPERTS), lambda i: (i, 0)),
            pl.BlockSpec((BM // 256, 256), lambda i: (i, 0)),
        ],
        out_shape=[
            jax.ShapeDtypeStruct((N_TOKENS, NUM_EXPERTS), jnp.float32),
            jax.ShapeDtypeStruct((N_TOKENS // 256, 256), jnp.int32),
        ],
        compiler_params=pltpu.CompilerParams(
            dimension_semantics=("parallel",),
        ),
    )(patch_feat, W1, b1, W2, b2)
    return probs, eid.reshape(N_TOKENS)


# speedup vs baseline: 1.7621x; 1.2551x over previous
import jax
import jax.numpy as jnp
from jax.experimental import pallas as pl
from jax.experimental.pallas import tpu as pltpu

N_TOKENS = 16384
IN_DIM = 1024
HIDDEN = 256
NUM_EXPERTS = 16

BM = 2048


def _dot(a, b):
    return jax.lax.dot_general(
        a, b, (((1,), (0,)), ((), ())), preferred_element_type=jnp.float32
    )


def _router_body(x_ref, w1_ref, b1_ref, w2_ref, b2_ref, probs_ref, eid_ref):
    w1 = w1_ref[...].astype(jnp.bfloat16)
    b1 = b1_ref[...].reshape(1, HIDDEN)
    w2 = w2_ref[...].astype(jnp.bfloat16)
    b2 = b2_ref[...].reshape(1, NUM_EXPERTS)
    h = _dot(x_ref[...].astype(jnp.bfloat16), w1)
    h = jnp.maximum(h + b1, 0.0)
    logits = _dot(h.astype(jnp.bfloat16), w2)
    logits = logits + b2
    m = jnp.max(logits, axis=-1, keepdims=True)
    e = jnp.exp(logits - m)
    probs = e / jnp.sum(e, axis=-1, keepdims=True)
    probs_ref[...] = jax.lax.transpose(probs, (1, 0))

    mask = (logits == m).astype(jnp.bfloat16)
    lt = (
        jax.lax.broadcasted_iota(jnp.int32, (NUM_EXPERTS, NUM_EXPERTS), 0)
        <= jax.lax.broadcasted_iota(jnp.int32, (NUM_EXPERTS, NUM_EXPERTS), 1)
    ).astype(jnp.bfloat16)
    cnt = _dot(mask, lt)
    first = jnp.where(cnt == 1.0, mask.astype(jnp.float32), 0.0).astype(jnp.bfloat16)
    first_t = jax.lax.transpose(first, (1, 0))
    iota_row = jax.lax.broadcasted_iota(
        jnp.int32, (1, NUM_EXPERTS), 1
    ).astype(jnp.bfloat16)
    eid_lane = _dot(iota_row, first_t)
    eid_ref[...] = eid_lane.astype(jnp.int32).reshape(BM)


def kernel(patch_feat, W1, b1, W2, b2):
    grid = (N_TOKENS // BM,)
    probs_t, eid = pl.pallas_call(
        _router_body,
        grid=grid,
        in_specs=[
            pl.BlockSpec((BM, IN_DIM), lambda i: (i, 0)),
            pl.BlockSpec((IN_DIM, HIDDEN), lambda i: (0, 0)),
            pl.BlockSpec((HIDDEN,), lambda i: (0,)),
            pl.BlockSpec((HIDDEN, NUM_EXPERTS), lambda i: (0, 0)),
            pl.BlockSpec((NUM_EXPERTS,), lambda i: (0,)),
        ],
        out_specs=[
            pl.BlockSpec((NUM_EXPERTS, BM), lambda i: (0, i)),
            pl.BlockSpec((BM,), lambda i: (i,)),
        ],
        out_shape=[
            jax.ShapeDtypeStruct((NUM_EXPERTS, N_TOKENS), jnp.float32),
            jax.ShapeDtypeStruct((N_TOKENS,), jnp.int32),
        ],
        compiler_params=pltpu.CompilerParams(
            dimension_semantics=("parallel",),
        ),
    )(patch_feat, W1, b1, W2, b2)
    return probs_t.T, eid
